# Initial kernel scaffold; baseline (speedup 1.0000x reference)
import jax
import jax.numpy as jnp
from jax.experimental import pallas as pl


def kernel(feature1, coord1, feature2, coord2, W1, b1, g1, be1, W2, b2, g2, be2):
    raise NotImplementedError("write your pallas kernel here")


# speedup vs baseline: 8.5421x; 8.5421x over previous
import functools

import jax
import jax.numpy as jnp
from jax import lax
from jax.experimental import pallas as pl
from jax.experimental.pallas import tpu as pltpu
from jax.experimental.pallas import tpu_sc as plsc

_B, _N, _S = 16, 4096, 1024
_C1, _C2 = 256, 512
_TNK = 512
_TNM = 512
_NW = 32
_CH = 32


def _knn_body(c1t_ref, c2_ref, gidx_ref, w0_ref, w1_ref, w2_ref):
    b = pl.program_id(0)
    c1 = c1t_ref[0]
    c2 = c2_ref[0]
    prod = lax.dot_general(c1, c2, (((1,), (0,)), ((), ())),
                           preferred_element_type=jnp.float32)
    sq1 = jnp.sum(c1 * c1, axis=1, keepdims=True)
    sq2 = jnp.sum(c2 * c2, axis=0, keepdims=True)
    d = sq1 - 2.0 * prod + sq2
    lanes = lax.broadcasted_iota(jnp.int32, d.shape, 1)
    big = jnp.int32(2 ** 30)
    idxs, vals = [], []
    for k in range(3):
        m = jnp.min(d, axis=1, keepdims=True)
        i = jnp.min(jnp.where(d == m, lanes, big), axis=1, keepdims=True)
        idxs.append(i)
        vals.append(m)
        if k < 2:
            d = jnp.where(lanes == i, jnp.float32(jnp.inf), d)
    r0 = 1.0 / (vals[0] + 1e-8)
    r1 = 1.0 / (vals[1] + 1e-8)
    r2 = 1.0 / (vals[2] + 1e-8)
    norm = r0 + r1 + r2
    gidx_ref[0] = jnp.concatenate(idxs, axis=1) + b * _S
    shp = (c1.shape[0], 16)
    w0_ref[0] = jnp.broadcast_to(r0 / norm, shp)
    w1_ref[0] = jnp.broadcast_to(r1 / norm, shp)
    w2_ref[0] = jnp.broadcast_to(r2 / norm, shp)


def _knn(c1t, coord2):
    B, N, _ = c1t.shape
    S = coord2.shape[2]
    nt = N // _TNK
    return pl.pallas_call(
        _knn_body,
        grid=(B, nt),
        in_specs=[
            pl.BlockSpec((1, _TNK, 3), lambda b, i: (b, i, 0)),
            pl.BlockSpec((1, 3, S), lambda b, i: (b, 0, 0)),
        ],
        out_specs=[
            pl.BlockSpec((1, _TNK, 3), lambda b, i: (b, i, 0)),
            pl.BlockSpec((1, _TNK, 16), lambda b, i: (b, i, 0)),
            pl.BlockSpec((1, _TNK, 16), lambda b, i: (b, i, 0)),
            pl.BlockSpec((1, _TNK, 16), lambda b, i: (b, i, 0)),
        ],
        out_shape=[
            jax.ShapeDtypeStruct((B, N, 3), jnp.int32),
            jax.ShapeDtypeStruct((B, N, 16), jnp.float32),
            jax.ShapeDtypeStruct((B, N, 16), jnp.float32),
            jax.ShapeDtypeStruct((B, N, 16), jnp.float32),
        ],
    )(c1t, coord2)


def _transpose_body(f2_ref, out_ref):
    out_ref[0] = f2_ref[0].T


def _transpose_f2(feature2):
    B, C2, S = feature2.shape
    return pl.pallas_call(
        _transpose_body,
        grid=(B,),
        in_specs=[pl.BlockSpec((1, C2, S), lambda b: (b, 0, 0))],
        out_specs=pl.BlockSpec((1, S, C2), lambda b: (b, 0, 0)),
        out_shape=jax.ShapeDtypeStruct((B, S, C2), jnp.float32),
    )(feature2)


def _interp_call(f2t, gidx, w0, w1, w2):
    P = w0.shape[0]
    C2 = f2t.shape[1]
    pw = P // _NW
    nch = pw // _CH
    mesh = plsc.VectorSubcoreMesh(core_axis_name="c", subcore_axis_name="s")

    @functools.partial(
        pl.kernel,
        out_type=jax.ShapeDtypeStruct((P, C2), jnp.float32),
        mesh=mesh,
        scratch_types=[
            pltpu.VMEM((_CH * 3,), jnp.int32),
            pltpu.VMEM((_CH * 3, C2), jnp.float32),
            pltpu.VMEM((_CH, 16), jnp.float32),
            pltpu.VMEM((_CH, 16), jnp.float32),
            pltpu.VMEM((_CH, 16), jnp.float32),
            pltpu.VMEM((_CH, C2), jnp.float32),
            pltpu.SemaphoreType.DMA,
        ],
    )
    def interp_k(f2t_hbm, gidx_hbm, w0_hbm, w1_hbm, w2_hbm, out_hbm,
                 idx_v, rows_v, w0_v, w1_v, w2_v, out_v, sem):
        wid = lax.axis_index("s") * 2 + lax.axis_index("c")

        def chunk(c, _):
            base = wid * pw + c * _CH
            pltpu.sync_copy(gidx_hbm.at[pl.ds(base * 3, _CH * 3)], idx_v)
            pltpu.async_copy(f2t_hbm.at[idx_v], rows_v, sem).wait()
            pltpu.sync_copy(w0_hbm.at[pl.ds(base, _CH)], w0_v)
            pltpu.sync_copy(w1_hbm.at[pl.ds(base, _CH)], w1_v)
            pltpu.sync_copy(w2_hbm.at[pl.ds(base, _CH)], w2_v)

            def point(p, _):
                a0 = w0_v[p]
                a1 = w1_v[p]
                a2 = w2_v[p]
                for g in range(C2 // 16):
                    sl = pl.ds(g * 16, 16)
                    out_v[p, sl] = (a0 * rows_v[3 * p, sl]
                                    + a1 * rows_v[3 * p + 1, sl]
                                    + a2 * rows_v[3 * p + 2, sl])
                return 0

            lax.fori_loop(0, _CH, point, 0)
            pltpu.sync_copy(out_v, out_hbm.at[pl.ds(base, _CH)])
            return 0

        lax.fori_loop(0, nch, chunk, 0)

    return interp_k(f2t, gidx, w0, w1, w2)


def _mlp1_body(f1_ref, itp_ref, w1a_ref, w1b_ref, b1_ref,
               y_ref, ps_ref, pq_ref):
    f1b = f1_ref[0]
    itp = itp_ref[0]
    ya = lax.dot_general(w1a_ref[...], f1b, (((1,), (0,)), ((), ())),
                         preferred_element_type=jnp.float32)
    yb = lax.dot_general(w1b_ref[...], itp, (((1,), (1,)), ((), ())),
                         preferred_element_type=jnp.float32)
    y = ya + yb + b1_ref[...]
    y_ref[0] = y
    ps_ref[0, 0] = jnp.sum(y, axis=1, keepdims=True)
    pq_ref[0, 0] = jnp.sum(y * y, axis=1, keepdims=True)


def _mlp1(feature1, interp, W1a, W1b, b1c):
    B, C1, N = feature1.shape
    C2 = interp.shape[2]
    d1 = W1a.shape[0]
    nt = N // _TNM
    return pl.pallas_call(
        _mlp1_body,
        grid=(B, nt),
        in_specs=[
            pl.BlockSpec((1, C1, _TNM), lambda b, i: (b, 0, i)),
            pl.BlockSpec((1, _TNM, C2), lambda b, i: (b, i, 0)),
            pl.BlockSpec((d1, C1), lambda b, i: (0, 0)),
            pl.BlockSpec((d1, C2), lambda b, i: (0, 0)),
            pl.BlockSpec((d1, 1), lambda b, i: (0, 0)),
        ],
        out_specs=[
            pl.BlockSpec((1, d1, _TNM), lambda b, i: (b, 0, i)),
            pl.BlockSpec((1, 1, d1, 1), lambda b, i: (b, i, 0, 0)),
            pl.BlockSpec((1, 1, d1, 1), lambda b, i: (b, i, 0, 0)),
        ],
        out_shape=[
            jax.ShapeDtypeStruct((B, d1, N), jnp.float32),
            jax.ShapeDtypeStruct((B, nt, d1, 1), jnp.float32),
            jax.ShapeDtypeStruct((B, nt, d1, 1), jnp.float32),
        ],
    )(feature1, interp, W1a, W1b, b1c)


def _mlp2_body(y1_ref, sc_ref, sh_ref, w2_ref, b2_ref,
               y_ref, ps_ref, pq_ref):
    z = jnp.maximum(y1_ref[0] * sc_ref[...] + sh_ref[...], 0.0)
    y = lax.dot_general(w2_ref[...], z, (((1,), (0,)), ((), ())),
                        preferred_element_type=jnp.float32) + b2_ref[...]
    y_ref[0] = y
    ps_ref[0, 0] = jnp.sum(y, axis=1, keepdims=True)
    pq_ref[0, 0] = jnp.sum(y * y, axis=1, keepdims=True)


def _mlp2(y1, scale1, shift1, W2, b2c):
    B, d1, N = y1.shape
    d2 = W2.shape[0]
    nt = N // _TNM
    return pl.pallas_call(
        _mlp2_body,
        grid=(B, nt),
        in_specs=[
            pl.BlockSpec((1, d1, _TNM), lambda b, i: (b, 0, i)),
            pl.BlockSpec((d1, 1), lambda b, i: (0, 0)),
            pl.BlockSpec((d1, 1), lambda b, i: (0, 0)),
            pl.BlockSpec((d2, d1), lambda b, i: (0, 0)),
            pl.BlockSpec((d2, 1), lambda b, i: (0, 0)),
        ],
        out_specs=[
            pl.BlockSpec((1, d2, _TNM), lambda b, i: (b, 0, i)),
            pl.BlockSpec((1, 1, d2, 1), lambda b, i: (b, i, 0, 0)),
            pl.BlockSpec((1, 1, d2, 1), lambda b, i: (b, i, 0, 0)),
        ],
        out_shape=[
            jax.ShapeDtypeStruct((B, d2, N), jnp.float32),
            jax.ShapeDtypeStruct((B, nt, d2, 1), jnp.float32),
            jax.ShapeDtypeStruct((B, nt, d2, 1), jnp.float32),
        ],
    )(y1, scale1, shift1, W2, b2c)


def _finish_body(y2_ref, sc_ref, sh_ref, out_ref):
    out_ref[0] = jnp.maximum(y2_ref[0] * sc_ref[...] + sh_ref[...], 0.0)


def _finish(y2, scale2, shift2):
    B, d2, N = y2.shape
    nt = N // _TNM
    return pl.pallas_call(
        _finish_body,
        grid=(B, nt),
        in_specs=[
            pl.BlockSpec((1, d2, _TNM), lambda b, i: (b, 0, i)),
            pl.BlockSpec((d2, 1), lambda b, i: (0, 0)),
            pl.BlockSpec((d2, 1), lambda b, i: (0, 0)),
        ],
        out_specs=pl.BlockSpec((1, d2, _TNM), lambda b, i: (b, 0, i)),
        out_shape=jax.ShapeDtypeStruct((B, d2, N), jnp.float32),
    )(y2, scale2, shift2)


def _stats(ps, pq, g, be, count):
    s = jnp.sum(ps, axis=(0, 1))
    q = jnp.sum(pq, axis=(0, 1))
    mean = s / count
    var = q / count - mean * mean
    scale = g[:, None] * lax.rsqrt(var + 1e-5)
    shift = be[:, None] - mean * scale
    return scale, shift


def kernel(feature1, coord1, feature2, coord2, W1, b1, g1, be1, W2, b2, g2, be2):
    B, C1, N = feature1.shape
    _, C2, S = feature2.shape
    assert (B, C1, N, C2, S) == (_B, _C1, _N, _C2, _S)
    P = B * N
    d1 = W1.shape[0]
    d2 = W2.shape[0]

    c1t = jnp.transpose(coord1, (0, 2, 1))
    gidx, w0, w1, w2 = _knn(c1t, coord2)
    f2t = _transpose_f2(feature2)

    interp = _interp_call(
        f2t.reshape(B * S, C2),
        gidx.reshape(P * 3),
        w0.reshape(P, 16), w1.reshape(P, 16), w2.reshape(P, 16),
    )

    y1, ps1, pq1 = _mlp1(feature1, interp.reshape(B, N, C2),
                         W1[:, :C1], W1[:, C1:], b1[:, None])
    scale1, shift1 = _stats(ps1, pq1, g1, be1, float(P))
    y2, ps2, pq2 = _mlp2(y1, scale1, shift1, W2, b2[:, None])
    scale2, shift2 = _stats(ps2, pq2, g2, be2, float(P))
    return _finish(y2, scale2, shift2)


# speedup vs baseline: 10.6370x; 1.2452x over previous
import functools

import jax
import jax.numpy as jnp
from jax import lax
from jax.experimental import pallas as pl
from jax.experimental.pallas import tpu as pltpu
from jax.experimental.pallas import tpu_sc as plsc

_B, _N, _S = 16, 4096, 1024
_C1, _C2 = 256, 512
_TNK = 512
_TNM = 512
_NW = 32
_CH = 32


def _knn_body(c1t_ref, c2_ref, gi0_ref, gi1_ref, gi2_ref,
              w0_ref, w1_ref, w2_ref):
    b = pl.program_id(0)
    c1 = c1t_ref[0]
    c2 = c2_ref[0]
    prod = lax.dot_general(c1, c2, (((1,), (0,)), ((), ())),
                           preferred_element_type=jnp.float32)
    sq1 = jnp.sum(c1 * c1, axis=1, keepdims=True)
    sq2 = jnp.sum(c2 * c2, axis=0, keepdims=True)
    d = sq1 - 2.0 * prod + sq2
    lanes = lax.broadcasted_iota(jnp.int32, d.shape, 1)
    big = jnp.int32(2 ** 30)
    idxs, vals = [], []
    for k in range(3):
        m = jnp.min(d, axis=1, keepdims=True)
        i = jnp.min(jnp.where(d == m, lanes, big), axis=1, keepdims=True)
        idxs.append(i)
        vals.append(m)
        if k < 2:
            d = jnp.where(lanes == i, jnp.float32(jnp.inf), d)
    r0 = 1.0 / (vals[0] + 1e-8)
    r1 = 1.0 / (vals[1] + 1e-8)
    r2 = 1.0 / (vals[2] + 1e-8)
    norm = r0 + r1 + r2
    gi0_ref[0] = idxs[0] + b * _S
    gi1_ref[0] = idxs[1] + b * _S
    gi2_ref[0] = idxs[2] + b * _S
    shp = (c1.shape[0], 16)
    w0_ref[0] = jnp.broadcast_to(r0 / norm, shp)
    w1_ref[0] = jnp.broadcast_to(r1 / norm, shp)
    w2_ref[0] = jnp.broadcast_to(r2 / norm, shp)


def _knn(c1t, coord2):
    B, N, _ = c1t.shape
    S = coord2.shape[2]
    nt = N // _TNK
    return pl.pallas_call(
        _knn_body,
        grid=(B, nt),
        in_specs=[
            pl.BlockSpec((1, _TNK, 3), lambda b, i: (b, i, 0)),
            pl.BlockSpec((1, 3, S), lambda b, i: (b, 0, 0)),
        ],
        out_specs=[
            pl.BlockSpec((1, _TNK, 1), lambda b, i: (b, i, 0)),
            pl.BlockSpec((1, _TNK, 1), lambda b, i: (b, i, 0)),
            pl.BlockSpec((1, _TNK, 1), lambda b, i: (b, i, 0)),
            pl.BlockSpec((1, _TNK, 16), lambda b, i: (b, i, 0)),
            pl.BlockSpec((1, _TNK, 16), lambda b, i: (b, i, 0)),
            pl.BlockSpec((1, _TNK, 16), lambda b, i: (b, i, 0)),
        ],
        out_shape=[
            jax.ShapeDtypeStruct((B, N, 1), jnp.int32),
            jax.ShapeDtypeStruct((B, N, 1), jnp.int32),
            jax.ShapeDtypeStruct((B, N, 1), jnp.int32),
            jax.ShapeDtypeStruct((B, N, 16), jnp.float32),
            jax.ShapeDtypeStruct((B, N, 16), jnp.float32),
            jax.ShapeDtypeStruct((B, N, 16), jnp.float32),
        ],
    )(c1t, coord2)


def _transpose_body(f2_ref, out_ref):
    out_ref[0] = f2_ref[0].T


def _transpose_f2(feature2):
    B, C2, S = feature2.shape
    return pl.pallas_call(
        _transpose_body,
        grid=(B,),
        in_specs=[pl.BlockSpec((1, C2, S), lambda b: (b, 0, 0))],
        out_specs=pl.BlockSpec((1, S, C2), lambda b: (b, 0, 0)),
        out_shape=jax.ShapeDtypeStruct((B, S, C2), jnp.float32),
    )(feature2)


def _interp_call(f2t, gi0, gi1, gi2, w0, w1, w2):
    P = w0.shape[0]
    C2 = f2t.shape[1]
    pw = P // _NW
    nch = pw // _CH
    mesh = plsc.VectorSubcoreMesh(core_axis_name="c", subcore_axis_name="s")

    @functools.partial(
        pl.kernel,
        out_type=jax.ShapeDtypeStruct((P, C2), jnp.float32),
        mesh=mesh,
        scratch_types=[
            pltpu.VMEM((2, 3, _CH), jnp.int32),
            pltpu.VMEM((2, _CH, C2), jnp.float32),
            pltpu.VMEM((2, _CH, C2), jnp.float32),
            pltpu.VMEM((2, _CH, C2), jnp.float32),
            pltpu.VMEM((2, _CH, 16), jnp.float32),
            pltpu.VMEM((2, _CH, 16), jnp.float32),
            pltpu.VMEM((2, _CH, 16), jnp.float32),
            pltpu.SemaphoreType.DMA,
            pltpu.SemaphoreType.DMA,
        ],
    )
    def interp_k(f2t_hbm, gi0_hbm, gi1_hbm, gi2_hbm, w0_hbm, w1_hbm, w2_hbm,
                 out_hbm, idx_v, r0_v, r1_v, r2_v, w0_v, w1_v, w2_v,
                 sem0, sem1):
        wid = lax.axis_index("s") * 2 + lax.axis_index("c")
        sems = (sem0, sem1)
        gis = (gi0_hbm, gi1_hbm, gi2_hbm)
        ws = (w0_hbm, w1_hbm, w2_hbm)
        rows = (r0_v, r1_v, r2_v)
        wv = (w0_v, w1_v, w2_v)

        def issue(c, buf):
            base = wid * pw + c * _CH
            for k in range(3):
                pltpu.sync_copy(gis[k].at[pl.ds(base, _CH)],
                                idx_v.at[buf, k])
                pltpu.async_copy(f2t_hbm.at[idx_v.at[buf, k]],
                                 rows[k].at[buf], sems[buf])
                pltpu.sync_copy(ws[k].at[pl.ds(base, _CH)], wv[k].at[buf])

        def compute(c, buf):
            base = wid * pw + c * _CH
            for k in range(3):
                pltpu.make_async_copy(f2t_hbm.at[idx_v.at[buf, k]],
                                      rows[k].at[buf], sems[buf]).wait()

            def point(p, _):
                a0 = w0_v[buf, p]
                a1 = w1_v[buf, p]
                a2 = w2_v[buf, p]
                for g in range(C2 // 16):
                    sl = pl.ds(g * 16, 16)
                    r0_v[buf, p, sl] = (a0 * r0_v[buf, p, sl]
                                        + a1 * r1_v[buf, p, sl]
                                        + a2 * r2_v[buf, p, sl])
                return 0

            lax.fori_loop(0, _CH, point, 0)
            pltpu.sync_copy(r0_v.at[buf], out_hbm.at[pl.ds(base, _CH)])

        issue(0, 0)

        def pair(t, _):
            c0 = 2 * t
            issue(c0 + 1, 1)
            compute(c0, 0)

            @pl.when(c0 + 2 < nch)
            def _():
                issue(c0 + 2, 0)

            compute(c0 + 1, 1)
            return 0

        lax.fori_loop(0, nch // 2, pair, 0)

    return interp_k(f2t, gi0, gi1, gi2, w0, w1, w2)


def _mlp1_body(f1_ref, itp_ref, w1a_ref, w1b_ref, b1_ref,
               y_ref, ps_ref, pq_ref):
    f1b = f1_ref[0]
    itp = itp_ref[0]
    ya = lax.dot_general(w1a_ref[...], f1b, (((1,), (0,)), ((), ())),
                         preferred_element_type=jnp.float32)
    yb = lax.dot_general(w1b_ref[...], itp, (((1,), (1,)), ((), ())),
                         preferred_element_type=jnp.float32)
    y = ya + yb + b1_ref[...]
    y_ref[0] = y
    ps_ref[0, 0] = jnp.sum(y, axis=1, keepdims=True)
    pq_ref[0, 0] = jnp.sum(y * y, axis=1, keepdims=True)


def _mlp1(feature1, interp, W1a, W1b, b1c):
    B, C1, N = feature1.shape
    C2 = interp.shape[2]
    d1 = W1a.shape[0]
    nt = N // _TNM
    return pl.pallas_call(
        _mlp1_body,
        grid=(B, nt),
        in_specs=[
            pl.BlockSpec((1, C1, _TNM), lambda b, i: (b, 0, i)),
            pl.BlockSpec((1, _TNM, C2), lambda b, i: (b, i, 0)),
            pl.BlockSpec((d1, C1), lambda b, i: (0, 0)),
            pl.BlockSpec((d1, C2), lambda b, i: (0, 0)),
            pl.BlockSpec((d1, 1), lambda b, i: (0, 0)),
        ],
        out_specs=[
            pl.BlockSpec((1, d1, _TNM), lambda b, i: (b, 0, i)),
            pl.BlockSpec((1, 1, d1, 1), lambda b, i: (b, i, 0, 0)),
            pl.BlockSpec((1, 1, d1, 1), lambda b, i: (b, i, 0, 0)),
        ],
        out_shape=[
            jax.ShapeDtypeStruct((B, d1, N), jnp.float32),
            jax.ShapeDtypeStruct((B, nt, d1, 1), jnp.float32),
            jax.ShapeDtypeStruct((B, nt, d1, 1), jnp.float32),
        ],
    )(feature1, interp, W1a, W1b, b1c)


def _mlp2_body(y1_ref, sc_ref, sh_ref, w2_ref, b2_ref,
               y_ref, ps_ref, pq_ref):
    z = jnp.maximum(y1_ref[0] * sc_ref[...] + sh_ref[...], 0.0)
    y = lax.dot_general(w2_ref[...], z, (((1,), (0,)), ((), ())),
                        preferred_element_type=jnp.float32) + b2_ref[...]
    y_ref[0] = y
    ps_ref[0, 0] = jnp.sum(y, axis=1, keepdims=True)
    pq_ref[0, 0] = jnp.sum(y * y, axis=1, keepdims=True)


def _mlp2(y1, scale1, shift1, W2, b2c):
    B, d1, N = y1.shape
    d2 = W2.shape[0]
    nt = N // _TNM
    return pl.pallas_call(
        _mlp2_body,
        grid=(B, nt),
        in_specs=[
            pl.BlockSpec((1, d1, _TNM), lambda b, i: (b, 0, i)),
            pl.BlockSpec((d1, 1), lambda b, i: (0, 0)),
            pl.BlockSpec((d1, 1), lambda b, i: (0, 0)),
            pl.BlockSpec((d2, d1), lambda b, i: (0, 0)),
            pl.BlockSpec((d2, 1), lambda b, i: (0, 0)),
        ],
        out_specs=[
            pl.BlockSpec((1, d2, _TNM), lambda b, i: (b, 0, i)),
            pl.BlockSpec((1, 1, d2, 1), lambda b, i: (b, i, 0, 0)),
            pl.BlockSpec((1, 1, d2, 1), lambda b, i: (b, i, 0, 0)),
        ],
        out_shape=[
            jax.ShapeDtypeStruct((B, d2, N), jnp.float32),
            jax.ShapeDtypeStruct((B, nt, d2, 1), jnp.float32),
            jax.ShapeDtypeStruct((B, nt, d2, 1), jnp.float32),
        ],
    )(y1, scale1, shift1, W2, b2c)


def _finish_body(y2_ref, sc_ref, sh_ref, out_ref):
    out_ref[0] = jnp.maximum(y2_ref[0] * sc_ref[...] + sh_ref[...], 0.0)


def _finish(y2, scale2, shift2):
    B, d2, N = y2.shape
    nt = N // _TNM
    return pl.pallas_call(
        _finish_body,
        grid=(B, nt),
        in_specs=[
            pl.BlockSpec((1, d2, _TNM), lambda b, i: (b, 0, i)),
            pl.BlockSpec((d2, 1), lambda b, i: (0, 0)),
            pl.BlockSpec((d2, 1), lambda b, i: (0, 0)),
        ],
        out_specs=pl.BlockSpec((1, d2, _TNM), lambda b, i: (b, 0, i)),
        out_shape=jax.ShapeDtypeStruct((B, d2, N), jnp.float32),
    )(y2, scale2, shift2)


def _stats(ps, pq, g, be, count):
    s = jnp.sum(ps, axis=(0, 1))
    q = jnp.sum(pq, axis=(0, 1))
    mean = s / count
    var = q / count - mean * mean
    scale = g[:, None] * lax.rsqrt(var + 1e-5)
    shift = be[:, None] - mean * scale
    return scale, shift


def kernel(feature1, coord1, feature2, coord2, W1, b1, g1, be1, W2, b2, g2, be2):
    B, C1, N = feature1.shape
    _, C2, S = feature2.shape
    assert (B, C1, N, C2, S) == (_B, _C1, _N, _C2, _S)
    P = B * N
    d1 = W1.shape[0]
    d2 = W2.shape[0]

    c1t = jnp.transpose(coord1, (0, 2, 1))
    gi0, gi1, gi2, w0, w1, w2 = _knn(c1t, coord2)
    f2t = _transpose_f2(feature2)

    interp = _interp_call(
        f2t.reshape(B * S, C2),
        gi0.reshape(P), gi1.reshape(P), gi2.reshape(P),
        w0.reshape(P, 16), w1.reshape(P, 16), w2.reshape(P, 16),
    )

    y1, ps1, pq1 = _mlp1(feature1, interp.reshape(B, N, C2),
                         W1[:, :C1], W1[:, C1:], b1[:, None])
    scale1, shift1 = _stats(ps1, pq1, g1, be1, float(P))
    y2, ps2, pq2 = _mlp2(y1, scale1, shift1, W2, b2[:, None])
    scale2, shift2 = _stats(ps2, pq2, g2, be2, float(P))
    return _finish(y2, scale2, shift2)


# speedup vs baseline: 11.4968x; 1.0808x over previous
import functools

import jax
import jax.numpy as jnp
from jax import lax
from jax.experimental import pallas as pl
from jax.experimental.pallas import tpu as pltpu
from jax.experimental.pallas import tpu_sc as plsc

_B, _N, _S = 16, 4096, 1024
_C1, _C2 = 256, 512
_TNK = 512
_TNM = 512
_NW = 32
_CH = 32


def _knn_body(c1t_ref, c2_ref, gi0_ref, gi1_ref, gi2_ref,
              w0_ref, w1_ref):
    b = pl.program_id(0)
    c1 = c1t_ref[0]
    c2 = c2_ref[0]
    prod = lax.dot_general(c1, c2, (((1,), (0,)), ((), ())),
                           preferred_element_type=jnp.float32)
    sq1 = jnp.sum(c1 * c1, axis=1, keepdims=True)
    sq2 = jnp.sum(c2 * c2, axis=0, keepdims=True)
    d = sq1 - 2.0 * prod + sq2
    lanes = lax.broadcasted_iota(jnp.int32, d.shape, 1)
    big = jnp.int32(2 ** 30)
    idxs, vals = [], []
    for k in range(3):
        m = jnp.min(d, axis=1, keepdims=True)
        i = jnp.min(jnp.where(d == m, lanes, big), axis=1, keepdims=True)
        idxs.append(i)
        vals.append(m)
        if k < 2:
            d = jnp.where(lanes == i, jnp.float32(jnp.inf), d)
    r0 = 1.0 / (vals[0] + 1e-8)
    r1 = 1.0 / (vals[1] + 1e-8)
    r2 = 1.0 / (vals[2] + 1e-8)
    norm = r0 + r1 + r2
    gi0_ref[0] = idxs[0] + b * _S
    gi1_ref[0] = idxs[1] + b * _S
    gi2_ref[0] = idxs[2] + b * _S
    shp = (c1.shape[0], 16)
    w0_ref[0] = jnp.broadcast_to(r0 / norm, shp)
    w1_ref[0] = jnp.broadcast_to(r1 / norm, shp)


def _knn(c1t, coord2):
    B, N, _ = c1t.shape
    S = coord2.shape[2]
    nt = N // _TNK
    return pl.pallas_call(
        _knn_body,
        grid=(B, nt),
        in_specs=[
            pl.BlockSpec((1, _TNK, 3), lambda b, i: (b, i, 0)),
            pl.BlockSpec((1, 3, S), lambda b, i: (b, 0, 0)),
        ],
        out_specs=[
            pl.BlockSpec((1, _TNK, 1), lambda b, i: (b, i, 0)),
            pl.BlockSpec((1, _TNK, 1), lambda b, i: (b, i, 0)),
            pl.BlockSpec((1, _TNK, 1), lambda b, i: (b, i, 0)),
            pl.BlockSpec((1, _TNK, 16), lambda b, i: (b, i, 0)),
            pl.BlockSpec((1, _TNK, 16), lambda b, i: (b, i, 0)),
        ],
        out_shape=[
            jax.ShapeDtypeStruct((B, N, 1), jnp.int32),
            jax.ShapeDtypeStruct((B, N, 1), jnp.int32),
            jax.ShapeDtypeStruct((B, N, 1), jnp.int32),
            jax.ShapeDtypeStruct((B, N, 16), jnp.float32),
            jax.ShapeDtypeStruct((B, N, 16), jnp.float32),
        ],
    )(c1t, coord2)


def _transpose_body(f2_ref, out_ref):
    y = f2_ref[0].T
    h = y.shape[1] // 2
    lo = lax.bitcast_convert_type(y[:, :h].astype(jnp.bfloat16), jnp.uint16)
    hi = lax.bitcast_convert_type(y[:, h:].astype(jnp.bfloat16), jnp.uint16)
    word = (lo.astype(jnp.uint32) | (hi.astype(jnp.uint32) << 16))
    out_ref[0] = lax.bitcast_convert_type(word, jnp.int32)


def _transpose_f2(feature2):
    B, C2, S = feature2.shape
    return pl.pallas_call(
        _transpose_body,
        grid=(B,),
        in_specs=[pl.BlockSpec((1, C2, S), lambda b: (b, 0, 0))],
        out_specs=pl.BlockSpec((1, S, C2 // 2), lambda b: (b, 0, 0)),
        out_shape=jax.ShapeDtypeStruct((B, S, C2 // 2), jnp.int32),
    )(feature2)


def _interp_call(f2t, gi0, gi1, gi2, w0, w1):
    P = w0.shape[0]
    CW = f2t.shape[1]
    pw = P // _NW
    nch = pw // _CH
    mesh = plsc.VectorSubcoreMesh(core_axis_name="c", subcore_axis_name="s")

    @functools.partial(
        pl.kernel,
        out_type=jax.ShapeDtypeStruct((P, CW), jnp.int32),
        mesh=mesh,
        scratch_types=[
            pltpu.VMEM((2, 3, _CH), jnp.int32),
            pltpu.VMEM((2, _CH, CW), jnp.int32),
            pltpu.VMEM((2, _CH, CW), jnp.int32),
            pltpu.VMEM((2, _CH, CW), jnp.int32),
            pltpu.VMEM((2, _CH, 16), jnp.float32),
            pltpu.VMEM((2, _CH, 16), jnp.float32),
            pltpu.SemaphoreType.DMA,
            pltpu.SemaphoreType.DMA,
        ],
    )
    def interp_k(f2t_hbm, gi0_hbm, gi1_hbm, gi2_hbm, w0_hbm, w1_hbm,
                 out_hbm, idx_v, r0_v, r1_v, r2_v, w0_v, w1_v,
                 sem0, sem1):
        wid = lax.axis_index("s") * 2 + lax.axis_index("c")
        sems = (sem0, sem1)
        gis = (gi0_hbm, gi1_hbm, gi2_hbm)
        ws = (w0_hbm, w1_hbm)
        rows = (r0_v, r1_v, r2_v)
        wv = (w0_v, w1_v)

        def issue(c, buf):
            base = wid * pw + c * _CH
            for k in range(3):
                pltpu.sync_copy(gis[k].at[pl.ds(base, _CH)],
                                idx_v.at[buf, k])
                pltpu.async_copy(f2t_hbm.at[idx_v.at[buf, k]],
                                 rows[k].at[buf], sems[buf])
            for k in range(2):
                pltpu.sync_copy(ws[k].at[pl.ds(base, _CH)], wv[k].at[buf])

        def compute(c, buf):
            base = wid * pw + c * _CH
            for k in range(3):
                pltpu.make_async_copy(f2t_hbm.at[idx_v.at[buf, k]],
                                      rows[k].at[buf], sems[buf]).wait()

            def point(p, _):
                f32 = jnp.float32
                i32 = jnp.int32
                hmask = jnp.int32(-65536)
                a0 = w0_v[buf, p]
                a1 = w1_v[buf, p]
                a2 = 1.0 - a0 - a1

                def halves(w):
                    lo = lax.bitcast_convert_type(w << 16, f32)
                    hi = lax.bitcast_convert_type(w & hmask, f32)
                    return lo, hi

                for g in range(CW // 16):
                    sl = pl.ds(g * 16, 16)
                    l0, h0 = halves(r0_v[buf, p, sl])
                    l1, h1 = halves(r1_v[buf, p, sl])
                    l2, h2 = halves(r2_v[buf, p, sl])
                    rnd = jnp.int32(32768)
                    alo = lax.bitcast_convert_type(
                        a0 * l0 + a1 * l1 + a2 * l2, i32) + rnd
                    ahi = lax.bitcast_convert_type(
                        a0 * h0 + a1 * h1 + a2 * h2, i32) + rnd
                    r0_v[buf, p, sl] = (
                        lax.shift_right_logical(alo, 16) | (ahi & hmask))
                return 0

            lax.fori_loop(0, _CH, point, 0)
            pltpu.sync_copy(r0_v.at[buf], out_hbm.at[pl.ds(base, _CH)])

        issue(0, 0)

        def pair(t, _):
            c0 = 2 * t
            issue(c0 + 1, 1)
            compute(c0, 0)

            @pl.when(c0 + 2 < nch)
            def _():
                issue(c0 + 2, 0)

            compute(c0 + 1, 1)
            return 0

        lax.fori_loop(0, nch // 2, pair, 0)

    return interp_k(f2t, gi0, gi1, gi2, w0, w1)


def _mlp1_body(f1_ref, itp_ref, w1a_ref, w1b_ref, b1_ref,
               y_ref, ps_ref, pq_ref):
    f1b = f1_ref[0].astype(jnp.bfloat16)
    w = lax.bitcast_convert_type(itp_ref[0], jnp.uint32)
    lo = lax.bitcast_convert_type(
        (w & 0xFFFF).astype(jnp.uint16), jnp.bfloat16)
    hi = lax.bitcast_convert_type(
        (w >> 16).astype(jnp.uint16), jnp.bfloat16)
    itp = jnp.concatenate([lo, hi], axis=1)
    ya = lax.dot_general(w1a_ref[...], f1b, (((1,), (0,)), ((), ())),
                         preferred_element_type=jnp.float32)
    yb = lax.dot_general(w1b_ref[...], itp, (((1,), (1,)), ((), ())),
                         preferred_element_type=jnp.float32)
    y = ya + yb + b1_ref[...]
    y_ref[0] = y
    ps_ref[0, 0] = jnp.sum(y, axis=1, keepdims=True)
    pq_ref[0, 0] = jnp.sum(y * y, axis=1, keepdims=True)


def _mlp1(feature1, interp, W1a, W1b, b1c):
    B, C1, N = feature1.shape
    C2 = interp.shape[2] * 2
    d1 = W1a.shape[0]
    nt = N // _TNM
    return pl.pallas_call(
        _mlp1_body,
        grid=(B, nt),
        in_specs=[
            pl.BlockSpec((1, C1, _TNM), lambda b, i: (b, 0, i)),
            pl.BlockSpec((1, _TNM, C2 // 2), lambda b, i: (b, i, 0)),
            pl.BlockSpec((d1, C1), lambda b, i: (0, 0)),
            pl.BlockSpec((d1, C2), lambda b, i: (0, 0)),
            pl.BlockSpec((d1, 1), lambda b, i: (0, 0)),
        ],
        out_specs=[
            pl.BlockSpec((1, d1, _TNM), lambda b, i: (b, 0, i)),
            pl.BlockSpec((1, 1, d1, 1), lambda b, i: (b, i, 0, 0)),
            pl.BlockSpec((1, 1, d1, 1), lambda b, i: (b, i, 0, 0)),
        ],
        out_shape=[
            jax.ShapeDtypeStruct((B, d1, N), jnp.float32),
            jax.ShapeDtypeStruct((B, nt, d1, 1), jnp.float32),
            jax.ShapeDtypeStruct((B, nt, d1, 1), jnp.float32),
        ],
    )(feature1, interp, W1a, W1b, b1c)


def _mlp2_body(y1_ref, sc_ref, sh_ref, w2_ref, b2_ref,
               y_ref, ps_ref, pq_ref):
    z = jnp.maximum(y1_ref[0] * sc_ref[...] + sh_ref[...], 0.0)
    y = lax.dot_general(w2_ref[...], z.astype(jnp.bfloat16),
                        (((1,), (0,)), ((), ())),
                        preferred_element_type=jnp.float32) + b2_ref[...]
    y_ref[0] = y
    ps_ref[0, 0] = jnp.sum(y, axis=1, keepdims=True)
    pq_ref[0, 0] = jnp.sum(y * y, axis=1, keepdims=True)


def _mlp2(y1, scale1, shift1, W2, b2c):
    B, d1, N = y1.shape
    d2 = W2.shape[0]
    nt = N // _TNM
    return pl.pallas_call(
        _mlp2_body,
        grid=(B, nt),
        in_specs=[
            pl.BlockSpec((1, d1, _TNM), lambda b, i: (b, 0, i)),
            pl.BlockSpec((d1, 1), lambda b, i: (0, 0)),
            pl.BlockSpec((d1, 1), lambda b, i: (0, 0)),
            pl.BlockSpec((d2, d1), lambda b, i: (0, 0)),
            pl.BlockSpec((d2, 1), lambda b, i: (0, 0)),
        ],
        out_specs=[
            pl.BlockSpec((1, d2, _TNM), lambda b, i: (b, 0, i)),
            pl.BlockSpec((1, 1, d2, 1), lambda b, i: (b, i, 0, 0)),
            pl.BlockSpec((1, 1, d2, 1), lambda b, i: (b, i, 0, 0)),
        ],
        out_shape=[
            jax.ShapeDtypeStruct((B, d2, N), jnp.float32),
            jax.ShapeDtypeStruct((B, nt, d2, 1), jnp.float32),
            jax.ShapeDtypeStruct((B, nt, d2, 1), jnp.float32),
        ],
    )(y1, scale1, shift1, W2, b2c)


def _finish_body(y2_ref, sc_ref, sh_ref, out_ref):
    out_ref[0] = jnp.maximum(y2_ref[0] * sc_ref[...] + sh_ref[...], 0.0)


def _finish(y2, scale2, shift2):
    B, d2, N = y2.shape
    nt = N // _TNM
    return pl.pallas_call(
        _finish_body,
        grid=(B, nt),
        in_specs=[
            pl.BlockSpec((1, d2, _TNM), lambda b, i: (b, 0, i)),
            pl.BlockSpec((d2, 1), lambda b, i: (0, 0)),
            pl.BlockSpec((d2, 1), lambda b, i: (0, 0)),
        ],
        out_specs=pl.BlockSpec((1, d2, _TNM), lambda b, i: (b, 0, i)),
        out_shape=jax.ShapeDtypeStruct((B, d2, N), jnp.float32),
    )(y2, scale2, shift2)


def _stats(ps, pq, g, be, count):
    s = jnp.sum(ps, axis=(0, 1))
    q = jnp.sum(pq, axis=(0, 1))
    mean = s / count
    var = q / count - mean * mean
    scale = g[:, None] * lax.rsqrt(var + 1e-5)
    shift = be[:, None] - mean * scale
    return scale, shift


def kernel(feature1, coord1, feature2, coord2, W1, b1, g1, be1, W2, b2, g2, be2):
    B, C1, N = feature1.shape
    _, C2, S = feature2.shape
    assert (B, C1, N, C2, S) == (_B, _C1, _N, _C2, _S)
    P = B * N
    d1 = W1.shape[0]
    d2 = W2.shape[0]

    c1t = jnp.transpose(coord1, (0, 2, 1))
    gi0, gi1, gi2, w0, w1 = _knn(c1t, coord2)
    f2t = _transpose_f2(feature2)

    interp_w = _interp_call(
        f2t.reshape(B * S, C2 // 2),
        gi0.reshape(P), gi1.reshape(P), gi2.reshape(P),
        w0.reshape(P, 16), w1.reshape(P, 16),
    )

    y1, ps1, pq1 = _mlp1(feature1, interp_w.reshape(B, N, C2 // 2),
                         W1[:, :C1].astype(jnp.bfloat16),
                         W1[:, C1:].astype(jnp.bfloat16), b1[:, None])
    scale1, shift1 = _stats(ps1, pq1, g1, be1, float(P))
    y2, ps2, pq2 = _mlp2(y1, scale1, shift1, W2.astype(jnp.bfloat16),
                         b2[:, None])
    scale2, shift2 = _stats(ps2, pq2, g2, be2, float(P))
    return _finish(y2, scale2, shift2)
